# Initial kernel scaffold; baseline (speedup 1.0000x reference)
import jax
import jax.numpy as jnp
from jax.experimental import pallas as pl


def kernel(x, edge_index, W0, b0, W1, b1):
    raise NotImplementedError("write your pallas kernel here")


# speedup vs baseline: 6.6683x; 6.6683x over previous
import functools

import jax
import jax.numpy as jnp
from jax import lax
from jax.experimental import pallas as pl
from jax.experimental.pallas import tpu as pltpu
from jax.experimental.pallas import tpu_sc as plsc

N = 10000
E = 320000
D = 128
DP = 144
NC = 2
NS = 16
NW = NC * NS
CHUNK = 64
NCH = 160
EPT = NCH * CHUNK
NP = 10224
RPT = NP // NS

_mesh = plsc.VectorSubcoreMesh(
    core_axis_name="c", subcore_axis_name="s", num_cores=NC, num_subcores=NS
)


@functools.partial(
    pl.kernel,
    out_type=jax.ShapeDtypeStruct((NC, NP, DP), jnp.float32),
    mesh=_mesh,
    compiler_params=pltpu.CompilerParams(use_tc_tiling_on_sc=False),
    scratch_types=[
        pltpu.VMEM((NCH, CHUNK), jnp.int32),
        pltpu.VMEM((NCH, CHUNK), jnp.int32),
        pltpu.VMEM((CHUNK, DP), jnp.float32),
        pltpu.VMEM((CHUNK, DP), jnp.float32),
        pltpu.VMEM_SHARED((NP, DP), jnp.float32),
        pltpu.SemaphoreType.DMA,
        pltpu.SemaphoreType.DMA,
    ],
)
def _sc_aggregate(xp_hbm, col_hbm, row_hbm, out_hbm,
                  colv, rowv, buf0, buf1, acc, semA, semB):
    c = lax.axis_index("c")
    s = lax.axis_index("s")
    wid = c * NS + s

    pltpu.sync_copy(col_hbm.at[wid], colv)
    pltpu.sync_copy(row_hbm.at[wid], rowv)

    zeros16 = jnp.zeros((16,), jnp.float32)

    def _z(i, carry):
        r = i // (DP // 16)
        q = lax.rem(i, DP // 16)
        buf0[r, pl.ds(q * 16, 16)] = zeros16
        return carry

    lax.fori_loop(0, CHUNK * (DP // 16), _z, 0)
    for k in range(RPT // CHUNK):
        pltpu.sync_copy(buf0, acc.at[pl.ds(s * RPT + k * CHUNK, CHUNK)])
    rem = RPT % CHUNK
    if rem:
        pltpu.sync_copy(buf0.at[pl.ds(0, rem)],
                        acc.at[pl.ds(s * RPT + (RPT // CHUNK) * CHUNK, rem)])
    plsc.subcore_barrier()

    def _pair(g, carry):
        j0 = g * 2
        j1 = j0 + 1
        d0 = pltpu.async_copy(xp_hbm.at[colv.at[j0]], buf0, semA)
        d1 = pltpu.async_copy(xp_hbm.at[colv.at[j1]], buf1, semB)
        d0.wait()
        pltpu.sync_copy(buf0, acc.at[rowv.at[j0]], add=True)
        d1.wait()
        pltpu.sync_copy(buf1, acc.at[rowv.at[j1]], add=True)
        return carry

    lax.fori_loop(0, NCH // 2, _pair, 0)
    plsc.subcore_barrier()

    pltpu.sync_copy(acc.at[pl.ds(s * RPT, RPT)], out_hbm.at[c, pl.ds(s * RPT, RPT)])


def _tc_body(p0_ref, p1_ref, w0_ref, b0_ref, w1_ref, b1_ref, y_ref):
    agg = p0_ref[0] + p1_ref[0]
    feat = agg[:, :D]
    deg = agg[:, D:D + 1]
    has_edge = jnp.minimum(deg, 1.0)
    m = feat / jnp.maximum(deg, 1.0)
    h = jnp.dot(m, w0_ref[...], preferred_element_type=jnp.float32)
    h = jnp.maximum(h + has_edge * b0_ref[...], 0.0)
    y_ref[...] = (jnp.dot(h, w1_ref[...], preferred_element_type=jnp.float32)
                  + b1_ref[...])


RB = 2000


def kernel(x, edge_index, W0, b0, W1, b1):
    col = edge_index[1].astype(jnp.int32)
    row = edge_index[0].astype(jnp.int32)
    pad = NW * EPT - E
    pad_idx = jnp.arange(pad, dtype=jnp.int32)
    colp = jnp.concatenate([col, pad_idx % N]).reshape(NW, NCH, CHUNK)
    rowp = jnp.concatenate([row, N + pad_idx % (NP - N)]).reshape(NW, NCH, CHUNK)
    xp = jnp.concatenate([x, jnp.ones((N, DP - D), jnp.float32)], axis=1)

    partials = _sc_aggregate(xp, colp, rowp)

    y = pl.pallas_call(
        _tc_body,
        grid=(N // RB,),
        in_specs=[
            pl.BlockSpec((1, RB, DP), lambda i: (0, i, 0)),
            pl.BlockSpec((1, RB, DP), lambda i: (1, i, 0)),
            pl.BlockSpec((D, D), lambda i: (0, 0)),
            pl.BlockSpec((1, D), lambda i: (0, 0)),
            pl.BlockSpec((D, D), lambda i: (0, 0)),
            pl.BlockSpec((1, D), lambda i: (0, 0)),
        ],
        out_specs=pl.BlockSpec((RB, D), lambda i: (i, 0)),
        out_shape=jax.ShapeDtypeStruct((N, D), jnp.float32),
    )(partials, partials, W0, b0.reshape(1, D), W1, b1.reshape(1, D))
    return y


# speedup vs baseline: 6.7806x; 1.0168x over previous
import functools

import jax
import jax.numpy as jnp
from jax import lax
from jax.experimental import pallas as pl
from jax.experimental.pallas import tpu as pltpu
from jax.experimental.pallas import tpu_sc as plsc

N = 10000
E = 320000
D = 128
DP = 144
NC = 2
NS = 16
NW = NC * NS
CHUNK = 64
NCH = 160
EPT = NCH * CHUNK
NP = 10224
RPT = NP // NS

_mesh = plsc.VectorSubcoreMesh(
    core_axis_name="c", subcore_axis_name="s", num_cores=NC, num_subcores=NS
)


@functools.partial(
    pl.kernel,
    out_type=jax.ShapeDtypeStruct((NC, NP, DP), jnp.float32),
    mesh=_mesh,
    compiler_params=pltpu.CompilerParams(use_tc_tiling_on_sc=False),
    scratch_types=[
        pltpu.VMEM((NCH, CHUNK), jnp.int32),
        pltpu.VMEM((NCH, CHUNK), jnp.int32),
        pltpu.VMEM((CHUNK, DP), jnp.float32),
        pltpu.VMEM((CHUNK, DP), jnp.float32),
        pltpu.VMEM_SHARED((NP, DP), jnp.float32),
        pltpu.SemaphoreType.DMA,
        pltpu.SemaphoreType.DMA,
        pltpu.SemaphoreType.DMA,
        pltpu.SemaphoreType.DMA,
    ],
)
def _sc_aggregate(xp_hbm, col_hbm, row_hbm, out_hbm,
                  colv, rowv, buf0, buf1, acc, semA, semB, semC, semD):
    c = lax.axis_index("c")
    s = lax.axis_index("s")
    wid = c * NS + s

    pltpu.sync_copy(col_hbm.at[wid], colv)
    pltpu.sync_copy(row_hbm.at[wid], rowv)

    zeros16 = jnp.zeros((16,), jnp.float32)

    def _z(i, carry):
        r = i // (DP // 16)
        q = lax.rem(i, DP // 16)
        buf0[r, pl.ds(q * 16, 16)] = zeros16
        return carry

    lax.fori_loop(0, CHUNK * (DP // 16), _z, 0)
    for k in range(RPT // CHUNK):
        pltpu.sync_copy(buf0, acc.at[pl.ds(s * RPT + k * CHUNK, CHUNK)])
    rem = RPT % CHUNK
    if rem:
        pltpu.sync_copy(buf0.at[pl.ds(0, rem)],
                        acc.at[pl.ds(s * RPT + (RPT // CHUNK) * CHUNK, rem)])
    plsc.subcore_barrier()

    def _pair(g, carry):
        j0 = g * 2
        j1 = j0 + 1
        d0 = pltpu.async_copy(xp_hbm.at[colv.at[j0]], buf0, semA)
        d1 = pltpu.async_copy(xp_hbm.at[colv.at[j1]], buf1, semB)
        d0.wait()
        s0 = pltpu.make_async_copy(buf0, acc.at[rowv.at[j0]], semC)
        s0.start(add=True)
        d1.wait()
        s1 = pltpu.make_async_copy(buf1, acc.at[rowv.at[j1]], semD)
        s1.start(add=True)
        s0.wait()
        s1.wait()
        return carry

    lax.fori_loop(0, NCH // 2, _pair, 0)
    plsc.subcore_barrier()

    pltpu.sync_copy(acc.at[pl.ds(s * RPT, RPT)], out_hbm.at[c, pl.ds(s * RPT, RPT)])


def _tc_body(p0_ref, p1_ref, w0_ref, b0_ref, w1_ref, b1_ref, y_ref):
    agg = p0_ref[0] + p1_ref[0]
    feat = agg[:, :D]
    deg = agg[:, D:D + 1]
    has_edge = jnp.minimum(deg, 1.0)
    m = feat / jnp.maximum(deg, 1.0)
    h = jnp.dot(m, w0_ref[...], preferred_element_type=jnp.float32)
    h = jnp.maximum(h + has_edge * b0_ref[...], 0.0)
    y_ref[...] = (jnp.dot(h, w1_ref[...], preferred_element_type=jnp.float32)
                  + b1_ref[...])


RB = 2000


def kernel(x, edge_index, W0, b0, W1, b1):
    col = edge_index[1].astype(jnp.int32)
    row = edge_index[0].astype(jnp.int32)
    pad = NW * EPT - E
    pad_idx = jnp.arange(pad, dtype=jnp.int32)
    colp = jnp.concatenate([col, pad_idx % N]).reshape(NW, NCH, CHUNK)
    rowp = jnp.concatenate([row, N + pad_idx % (NP - N)]).reshape(NW, NCH, CHUNK)
    xp = jnp.concatenate([x, jnp.ones((N, DP - D), jnp.float32)], axis=1)

    partials = _sc_aggregate(xp, colp, rowp)

    y = pl.pallas_call(
        _tc_body,
        grid=(N // RB,),
        in_specs=[
            pl.BlockSpec((1, RB, DP), lambda i: (0, i, 0)),
            pl.BlockSpec((1, RB, DP), lambda i: (1, i, 0)),
            pl.BlockSpec((D, D), lambda i: (0, 0)),
            pl.BlockSpec((1, D), lambda i: (0, 0)),
            pl.BlockSpec((D, D), lambda i: (0, 0)),
            pl.BlockSpec((1, D), lambda i: (0, 0)),
        ],
        out_specs=pl.BlockSpec((RB, D), lambda i: (i, 0)),
        out_shape=jax.ShapeDtypeStruct((N, D), jnp.float32),
    )(partials, partials, W0, b0.reshape(1, D), W1, b1.reshape(1, D))
    return y


# speedup vs baseline: 7.8288x; 1.1546x over previous
import functools

import numpy as np
import jax
import jax.numpy as jnp
from jax import lax
from jax.experimental import pallas as pl
from jax.experimental.pallas import tpu as pltpu
from jax.experimental.pallas import tpu_sc as plsc

N = 10000
E = 320000
D = 128
DDEG = 16
NC = 2
NS = 16
NW = NC * NS
CHUNK = 64
NCH = 160
EPT = NCH * CHUNK
IDXW = 128
NIDX = EPT // IDXW
NP = 10224
RPT = NP // NS
OPT = N // NS

_mesh = plsc.VectorSubcoreMesh(
    core_axis_name="c", subcore_axis_name="s", num_cores=NC, num_subcores=NS
)


@functools.partial(
    pl.kernel,
    out_type=[
        jax.ShapeDtypeStruct((NC, N, D), jnp.float32),
        jax.ShapeDtypeStruct((NC, N, DDEG), jnp.float32),
    ],
    mesh=_mesh,
    compiler_params=pltpu.CompilerParams(use_tc_tiling_on_sc=False),
    scratch_types=[
        pltpu.VMEM((NIDX, IDXW), jnp.int32),
        pltpu.VMEM((NIDX, IDXW), jnp.int32),
        pltpu.VMEM((2, CHUNK, D), jnp.float32),
        pltpu.VMEM((CHUNK, DDEG), jnp.float32),
        pltpu.VMEM((CHUNK, DDEG), jnp.float32),
        pltpu.VMEM_SHARED((NP, D), jnp.float32),
        pltpu.VMEM_SHARED((NP, DDEG), jnp.float32),
        [pltpu.SemaphoreType.DMA] * 2,
        [pltpu.SemaphoreType.DMA] * 2,
        [pltpu.SemaphoreType.DMA] * 2,
    ],
)
def _sc_aggregate(x_hbm, col_hbm, row_hbm, outf_hbm, outd_hbm,
                  colv, rowv, bufs, onesb, zerob, accf, accd,
                  gsems, fsems, dsems):
    c = lax.axis_index("c")
    s = lax.axis_index("s")
    wid = c * NS + s

    pltpu.sync_copy(col_hbm.at[wid], colv)
    pltpu.sync_copy(row_hbm.at[wid], rowv)

    zeros16 = jnp.zeros((16,), jnp.float32)
    ones16 = jnp.ones((16,), jnp.float32)

    def _z(i, carry):
        r = i // (D // 16)
        q = lax.rem(i, D // 16)
        bufs[r // CHUNK, lax.rem(r, CHUNK), pl.ds(q * 16, 16)] = zeros16
        return carry

    lax.fori_loop(0, 2 * CHUNK * (D // 16), _z, 0)

    def _z2(i, carry):
        onesb[i, pl.ds(0, 16)] = ones16
        zerob[i, pl.ds(0, 16)] = zeros16
        return carry

    lax.fori_loop(0, CHUNK, _z2, 0)

    base = s * RPT
    for k in range(RPT // CHUNK):
        pltpu.sync_copy(bufs.at[0], accf.at[pl.ds(base + k * CHUNK, CHUNK)])
        pltpu.sync_copy(zerob, accd.at[pl.ds(base + k * CHUNK, CHUNK)])
    rem = RPT % CHUNK
    if rem:
        off = base + (RPT // CHUNK) * CHUNK
        pltpu.sync_copy(bufs.at[0].at[pl.ds(0, rem)], accf.at[pl.ds(off, rem)])
        pltpu.sync_copy(zerob.at[pl.ds(0, rem)], accd.at[pl.ds(off, rem)])
    plsc.subcore_barrier()

    def _pair(g, carry):
        cidx = [colv.at[g, pl.ds(b * CHUNK, CHUNK)] for b in range(2)]
        ridx = [rowv.at[g, pl.ds(b * CHUNK, CHUNK)] for b in range(2)]
        ds = [pltpu.async_copy(x_hbm.at[cidx[b]], bufs.at[b], gsems[b])
              for b in range(2)]
        ss = []
        for b in range(2):
            ds[b].wait()
            sf = pltpu.make_async_copy(bufs.at[b], accf.at[ridx[b]], fsems[b])
            sf.start(add=True)
            sd = pltpu.make_async_copy(onesb, accd.at[ridx[b]], dsems[b])
            sd.start(add=True)
            ss += [sf, sd]
        for sc_ in ss:
            sc_.wait()
        return carry

    lax.fori_loop(0, NIDX, _pair, 0)
    plsc.subcore_barrier()

    pltpu.sync_copy(accf.at[pl.ds(s * OPT, OPT)], outf_hbm.at[c, pl.ds(s * OPT, OPT)])
    pltpu.sync_copy(accd.at[pl.ds(s * OPT, OPT)], outd_hbm.at[c, pl.ds(s * OPT, OPT)])


def _tc_body(pf0_ref, pf1_ref, pd0_ref, pd1_ref,
             w0_ref, b0_ref, w1_ref, b1_ref, y_ref):
    feat = pf0_ref[0] + pf1_ref[0]
    deg = (pd0_ref[0] + pd1_ref[0])[:, :1]
    has_edge = jnp.minimum(deg, 1.0)
    m = feat / jnp.maximum(deg, 1.0)
    h = jnp.dot(m, w0_ref[...], preferred_element_type=jnp.float32)
    h = jnp.maximum(h + has_edge * b0_ref[...], 0.0)
    y_ref[...] = (jnp.dot(h, w1_ref[...], preferred_element_type=jnp.float32)
                  + b1_ref[...])


RB = 2000

_PAD = NW * EPT - E
_PAD_COL = np.arange(_PAD, dtype=np.int32) % N
_PAD_ROW = N + np.arange(_PAD, dtype=np.int32) % (NP - N)


def kernel(x, edge_index, W0, b0, W1, b1):
    col = edge_index[1].astype(jnp.int32)
    row = edge_index[0].astype(jnp.int32)
    colp = jnp.concatenate([col, jnp.asarray(_PAD_COL)]).reshape(NW, NIDX, IDXW)
    rowp = jnp.concatenate([row, jnp.asarray(_PAD_ROW)]).reshape(NW, NIDX, IDXW)

    pf, pd = _sc_aggregate(x, colp, rowp)

    y = pl.pallas_call(
        _tc_body,
        grid=(N // RB,),
        in_specs=[
            pl.BlockSpec((1, RB, D), lambda i: (0, i, 0)),
            pl.BlockSpec((1, RB, D), lambda i: (1, i, 0)),
            pl.BlockSpec((1, RB, DDEG), lambda i: (0, i, 0)),
            pl.BlockSpec((1, RB, DDEG), lambda i: (1, i, 0)),
            pl.BlockSpec((D, D), lambda i: (0, 0)),
            pl.BlockSpec((1, D), lambda i: (0, 0)),
            pl.BlockSpec((D, D), lambda i: (0, 0)),
            pl.BlockSpec((1, D), lambda i: (0, 0)),
        ],
        out_specs=pl.BlockSpec((RB, D), lambda i: (i, 0)),
        out_shape=jax.ShapeDtypeStruct((N, D), jnp.float32),
    )(pf, pf, pd, pd, W0, b0.reshape(1, D), W1, b1.reshape(1, D))
    return y


# speedup vs baseline: 9.1162x; 1.1645x over previous
import functools

import numpy as np
import jax
import jax.numpy as jnp
from jax import lax
from jax.experimental import pallas as pl
from jax.experimental.pallas import tpu as pltpu
from jax.experimental.pallas import tpu_sc as plsc

N = 10000
E = 320000
D = 128
DDEG = 16
NC = 2
NS = 16
NW = NC * NS
CHUNK = 64
NCH = 160
EPT = NCH * CHUNK
IDXW = 128
NIDX = EPT // IDXW
NP = 10224
RPT = NP // NS
OPT = N // NS

_mesh = plsc.VectorSubcoreMesh(
    core_axis_name="c", subcore_axis_name="s", num_cores=NC, num_subcores=NS
)


@functools.partial(
    pl.kernel,
    out_type=[
        jax.ShapeDtypeStruct((NC, N, D), jnp.bfloat16),
        jax.ShapeDtypeStruct((NC, N, DDEG), jnp.float32),
    ],
    mesh=_mesh,
    compiler_params=pltpu.CompilerParams(use_tc_tiling_on_sc=False),
    scratch_types=[
        pltpu.VMEM((NIDX, IDXW), jnp.int32),
        pltpu.VMEM((NIDX, IDXW), jnp.int32),
        pltpu.VMEM((2, CHUNK, D), jnp.bfloat16),
        pltpu.VMEM((CHUNK, DDEG), jnp.float32),
        pltpu.VMEM((CHUNK, DDEG), jnp.float32),
        pltpu.VMEM_SHARED((NP, D), jnp.bfloat16),
        pltpu.VMEM_SHARED((NP, DDEG), jnp.float32),
        [pltpu.SemaphoreType.DMA] * 2,
        [pltpu.SemaphoreType.DMA] * 2,
        [pltpu.SemaphoreType.DMA] * 2,
    ],
)
def _sc_aggregate(x_hbm, col_hbm, row_hbm, outf_hbm, outd_hbm,
                  colv, rowv, bufs, onesb, zerob, accf, accd,
                  gsems, fsems, dsems):
    c = lax.axis_index("c")
    s = lax.axis_index("s")
    wid = c * NS + s

    pltpu.sync_copy(col_hbm.at[wid], colv)
    pltpu.sync_copy(row_hbm.at[wid], rowv)

    zeros16 = jnp.zeros((16,), jnp.float32)
    ones16 = jnp.ones((16,), jnp.float32)
    zeros32b = jnp.zeros((32,), jnp.bfloat16)

    def _z(i, carry):
        q = lax.rem(i, D // 32)
        bufs[0, i // (D // 32), pl.ds(q * 32, 32)] = zeros32b
        return carry

    lax.fori_loop(0, CHUNK * (D // 32), _z, 0)

    def _z2(i, carry):
        onesb[i, pl.ds(0, 16)] = ones16
        zerob[i, pl.ds(0, 16)] = zeros16
        return carry

    lax.fori_loop(0, CHUNK, _z2, 0)

    base = s * RPT
    for k in range(RPT // CHUNK):
        pltpu.sync_copy(bufs.at[0], accf.at[pl.ds(base + k * CHUNK, CHUNK)])
        pltpu.sync_copy(zerob, accd.at[pl.ds(base + k * CHUNK, CHUNK)])
    rem = RPT % CHUNK
    if rem:
        off = base + (RPT // CHUNK) * CHUNK
        pltpu.sync_copy(bufs.at[0].at[pl.ds(0, rem)], accf.at[pl.ds(off, rem)])
        pltpu.sync_copy(zerob.at[pl.ds(0, rem)], accd.at[pl.ds(off, rem)])
    plsc.subcore_barrier()

    def _pair(g, carry):
        cidx = [colv.at[g, pl.ds(b * CHUNK, CHUNK)] for b in range(2)]
        ridx = [rowv.at[g, pl.ds(b * CHUNK, CHUNK)] for b in range(2)]
        ds = [pltpu.async_copy(x_hbm.at[cidx[b]], bufs.at[b], gsems[b])
              for b in range(2)]
        ss = []
        for b in range(2):
            ds[b].wait()
            sf = pltpu.make_async_copy(bufs.at[b], accf.at[ridx[b]], fsems[b])
            sf.start(add=True)
            sd = pltpu.make_async_copy(onesb, accd.at[ridx[b]], dsems[b])
            sd.start(add=True)
            ss += [sf, sd]
        for sc_ in ss:
            sc_.wait()
        return carry

    lax.fori_loop(0, NIDX, _pair, 0)
    plsc.subcore_barrier()

    pltpu.sync_copy(accf.at[pl.ds(s * OPT, OPT)], outf_hbm.at[c, pl.ds(s * OPT, OPT)])
    pltpu.sync_copy(accd.at[pl.ds(s * OPT, OPT)], outd_hbm.at[c, pl.ds(s * OPT, OPT)])


def _tc_body(pf0_ref, pf1_ref, pd0_ref, pd1_ref,
             w0_ref, b0_ref, w1_ref, b1_ref, y_ref):
    feat = (pf0_ref[0].astype(jnp.float32)
            + pf1_ref[0].astype(jnp.float32))
    deg = (pd0_ref[0] + pd1_ref[0])[:, :1]
    has_edge = jnp.minimum(deg, 1.0)
    m = feat / jnp.maximum(deg, 1.0)
    h = jnp.dot(m, w0_ref[...], preferred_element_type=jnp.float32)
    h = jnp.maximum(h + has_edge * b0_ref[...], 0.0)
    y_ref[...] = (jnp.dot(h, w1_ref[...], preferred_element_type=jnp.float32)
                  + b1_ref[...])


RB = 2000

_PAD = NW * EPT - E
_PAD_COL = np.arange(_PAD, dtype=np.int32) % N
_PAD_ROW = N + np.arange(_PAD, dtype=np.int32) % (NP - N)


def kernel(x, edge_index, W0, b0, W1, b1):
    col = edge_index[1].astype(jnp.int32)
    row = edge_index[0].astype(jnp.int32)
    colp = jnp.concatenate([col, jnp.asarray(_PAD_COL)]).reshape(NW, NIDX, IDXW)
    rowp = jnp.concatenate([row, jnp.asarray(_PAD_ROW)]).reshape(NW, NIDX, IDXW)

    pf, pd = _sc_aggregate(x.astype(jnp.bfloat16), colp, rowp)

    y = pl.pallas_call(
        _tc_body,
        grid=(N // RB,),
        in_specs=[
            pl.BlockSpec((1, RB, D), lambda i: (0, i, 0)),
            pl.BlockSpec((1, RB, D), lambda i: (1, i, 0)),
            pl.BlockSpec((1, RB, DDEG), lambda i: (0, i, 0)),
            pl.BlockSpec((1, RB, DDEG), lambda i: (1, i, 0)),
            pl.BlockSpec((D, D), lambda i: (0, 0)),
            pl.BlockSpec((1, D), lambda i: (0, 0)),
            pl.BlockSpec((D, D), lambda i: (0, 0)),
            pl.BlockSpec((1, D), lambda i: (0, 0)),
        ],
        out_specs=pl.BlockSpec((RB, D), lambda i: (i, 0)),
        out_shape=jax.ShapeDtypeStruct((N, D), jnp.float32),
    )(pf, pf, pd, pd, W0, b0.reshape(1, D), W1, b1.reshape(1, D))
    return y


# speedup vs baseline: 9.5990x; 1.0530x over previous
import functools

import jax
import jax.numpy as jnp
from jax import lax
from jax.experimental import pallas as pl
from jax.experimental.pallas import tpu as pltpu
from jax.experimental.pallas import tpu_sc as plsc

N = 10000
E = 320000
D = 128
DDEG = 16
NC = 2
NS = 16
NW = NC * NS
EPT = E // NW
CHUNK = 80
NCH = EPT // CHUNK
OPT = N // NS

_mesh = plsc.VectorSubcoreMesh(
    core_axis_name="c", subcore_axis_name="s", num_cores=NC, num_subcores=NS
)


@functools.partial(
    pl.kernel,
    out_type=[
        jax.ShapeDtypeStruct((NC, N, D), jnp.bfloat16),
        jax.ShapeDtypeStruct((NC, N, DDEG), jnp.float32),
    ],
    mesh=_mesh,
    compiler_params=pltpu.CompilerParams(use_tc_tiling_on_sc=False),
    scratch_types=[
        pltpu.VMEM((EPT,), jnp.int32),
        pltpu.VMEM((EPT,), jnp.int32),
        pltpu.VMEM((2, CHUNK, D), jnp.bfloat16),
        pltpu.VMEM((CHUNK, DDEG), jnp.float32),
        pltpu.VMEM((CHUNK, DDEG), jnp.float32),
        pltpu.VMEM_SHARED((N, D), jnp.bfloat16),
        pltpu.VMEM_SHARED((N, DDEG), jnp.float32),
        [pltpu.SemaphoreType.DMA] * 2,
        [pltpu.SemaphoreType.DMA] * 2,
        [pltpu.SemaphoreType.DMA] * 2,
    ],
)
def _sc_aggregate(x_hbm, col_hbm, row_hbm, outf_hbm, outd_hbm,
                  colv, rowv, bufs, onesb, zerob, accf, accd,
                  gsems, fsems, dsems):
    c = lax.axis_index("c")
    s = lax.axis_index("s")
    wid = c * NS + s

    pltpu.sync_copy(col_hbm.at[wid], colv)
    pltpu.sync_copy(row_hbm.at[wid], rowv)

    zeros16 = jnp.zeros((16,), jnp.float32)
    ones16 = jnp.ones((16,), jnp.float32)
    zeros32b = jnp.zeros((32,), jnp.bfloat16)

    def _z(i, carry):
        q = lax.rem(i, D // 32)
        bufs[0, i // (D // 32), pl.ds(q * 32, 32)] = zeros32b
        return carry

    lax.fori_loop(0, CHUNK * (D // 32), _z, 0)

    def _z2(i, carry):
        onesb[i, pl.ds(0, 16)] = ones16
        zerob[i, pl.ds(0, 16)] = zeros16
        return carry

    lax.fori_loop(0, CHUNK, _z2, 0)

    base = s * OPT
    for k in range(OPT // CHUNK):
        pltpu.sync_copy(bufs.at[0], accf.at[pl.ds(base + k * CHUNK, CHUNK)])
        pltpu.sync_copy(zerob, accd.at[pl.ds(base + k * CHUNK, CHUNK)])
    rem = OPT % CHUNK
    if rem:
        off = base + (OPT // CHUNK) * CHUNK
        pltpu.sync_copy(bufs.at[0].at[pl.ds(0, rem)], accf.at[pl.ds(off, rem)])
        pltpu.sync_copy(zerob.at[pl.ds(0, rem)], accd.at[pl.ds(off, rem)])
    plsc.subcore_barrier()

    def _pair(g, carry):
        j0 = g * 2
        cidx = [colv.at[pl.ds((j0 + b) * CHUNK, CHUNK)] for b in range(2)]
        ridx = [rowv.at[pl.ds((j0 + b) * CHUNK, CHUNK)] for b in range(2)]
        ds = [pltpu.async_copy(x_hbm.at[cidx[b]], bufs.at[b], gsems[b])
              for b in range(2)]
        ss = []
        for b in range(2):
            ds[b].wait()
            sf = pltpu.make_async_copy(bufs.at[b], accf.at[ridx[b]], fsems[b])
            sf.start(add=True)
            sd = pltpu.make_async_copy(onesb, accd.at[ridx[b]], dsems[b])
            sd.start(add=True)
            ss += [sf, sd]
        for sc_ in ss:
            sc_.wait()
        return carry

    lax.fori_loop(0, NCH // 2, _pair, 0)
    jl = NCH - 1
    cl = colv.at[pl.ds(jl * CHUNK, CHUNK)]
    rl = rowv.at[pl.ds(jl * CHUNK, CHUNK)]
    pltpu.async_copy(x_hbm.at[cl], bufs.at[0], gsems[0]).wait()
    sf = pltpu.make_async_copy(bufs.at[0], accf.at[rl], fsems[0])
    sf.start(add=True)
    sd = pltpu.make_async_copy(onesb, accd.at[rl], dsems[0])
    sd.start(add=True)
    sf.wait()
    sd.wait()
    plsc.subcore_barrier()

    pltpu.sync_copy(accf.at[pl.ds(base, OPT)], outf_hbm.at[c, pl.ds(base, OPT)])
    pltpu.sync_copy(accd.at[pl.ds(base, OPT)], outd_hbm.at[c, pl.ds(base, OPT)])


def _tc_body(pf0_ref, pf1_ref, pd0_ref, pd1_ref,
             w0_ref, b0_ref, w1_ref, b1_ref, y_ref):
    feat = (pf0_ref[0].astype(jnp.float32)
            + pf1_ref[0].astype(jnp.float32))
    deg = (pd0_ref[0] + pd1_ref[0])[:, :1]
    has_edge = jnp.minimum(deg, 1.0)
    m = feat / jnp.maximum(deg, 1.0)
    h = jnp.dot(m, w0_ref[...], preferred_element_type=jnp.float32)
    h = jnp.maximum(h + has_edge * b0_ref[...], 0.0)
    y_ref[...] = (jnp.dot(h, w1_ref[...], preferred_element_type=jnp.float32)
                  + b1_ref[...])


RB = 2000


def kernel(x, edge_index, W0, b0, W1, b1):
    colp = edge_index[1].astype(jnp.int32).reshape(NW, EPT)
    rowp = edge_index[0].astype(jnp.int32).reshape(NW, EPT)

    pf, pd = _sc_aggregate(x.astype(jnp.bfloat16), colp, rowp)

    y = pl.pallas_call(
        _tc_body,
        grid=(N // RB,),
        in_specs=[
            pl.BlockSpec((1, RB, D), lambda i: (0, i, 0)),
            pl.BlockSpec((1, RB, D), lambda i: (1, i, 0)),
            pl.BlockSpec((1, RB, DDEG), lambda i: (0, i, 0)),
            pl.BlockSpec((1, RB, DDEG), lambda i: (1, i, 0)),
            pl.BlockSpec((D, D), lambda i: (0, 0)),
            pl.BlockSpec((1, D), lambda i: (0, 0)),
            pl.BlockSpec((D, D), lambda i: (0, 0)),
            pl.BlockSpec((1, D), lambda i: (0, 0)),
        ],
        out_specs=pl.BlockSpec((RB, D), lambda i: (i, 0)),
        out_shape=jax.ShapeDtypeStruct((N, D), jnp.float32),
    )(pf, pf, pd, pd, W0, b0.reshape(1, D), W1, b1.reshape(1, D))
    return y


# speedup vs baseline: 10.8025x; 1.1254x over previous
import functools

import jax
import jax.numpy as jnp
from jax import lax
from jax.experimental import pallas as pl
from jax.experimental.pallas import tpu as pltpu
from jax.experimental.pallas import tpu_sc as plsc

N = 10000
E = 320000
D = 128
DDEG = 16
NC = 2
NS = 16
NW = NC * NS
EPT = E // NW
CHUNK = 80
NCH = EPT // CHUNK
OPT = N // NS

_mesh = plsc.VectorSubcoreMesh(
    core_axis_name="c", subcore_axis_name="s", num_cores=NC, num_subcores=NS
)


@functools.partial(
    pl.kernel,
    out_type=[
        jax.ShapeDtypeStruct((NC, N, D), jnp.bfloat16),
        jax.ShapeDtypeStruct((NC, N, DDEG), jnp.float32),
    ],
    mesh=_mesh,
    compiler_params=pltpu.CompilerParams(use_tc_tiling_on_sc=False),
    scratch_types=[
        pltpu.VMEM((EPT,), jnp.int32),
        pltpu.VMEM((EPT,), jnp.int32),
        pltpu.VMEM((5, CHUNK, D), jnp.bfloat16),
        pltpu.VMEM((CHUNK, DDEG), jnp.float32),
        pltpu.VMEM((CHUNK, DDEG), jnp.float32),
        pltpu.VMEM_SHARED((N, D), jnp.bfloat16),
        pltpu.VMEM_SHARED((N, DDEG), jnp.float32),
        [pltpu.SemaphoreType.DMA] * 5,
        [pltpu.SemaphoreType.DMA] * 5,
        [pltpu.SemaphoreType.DMA] * 5,
    ],
)
def _sc_aggregate(x_hbm, col_hbm, row_hbm, outf_hbm, outd_hbm,
                  colv, rowv, bufs, onesb, zerob, accf, accd,
                  gsems, fsems, dsems):
    c = lax.axis_index("c")
    s = lax.axis_index("s")
    wid = c * NS + s

    pltpu.sync_copy(col_hbm.at[wid], colv)
    pltpu.sync_copy(row_hbm.at[wid], rowv)

    zeros16 = jnp.zeros((16,), jnp.float32)
    ones16 = jnp.ones((16,), jnp.float32)
    zeros32b = jnp.zeros((32,), jnp.bfloat16)

    def _z(i, carry):
        q = lax.rem(i, D // 32)
        bufs[0, i // (D // 32), pl.ds(q * 32, 32)] = zeros32b
        return carry

    lax.fori_loop(0, CHUNK * (D // 32), _z, 0)

    def _z2(i, carry):
        onesb[i, pl.ds(0, 16)] = ones16
        zerob[i, pl.ds(0, 16)] = zeros16
        return carry

    lax.fori_loop(0, CHUNK, _z2, 0)

    base = s * OPT
    for k in range(OPT // CHUNK):
        pltpu.sync_copy(bufs.at[0], accf.at[pl.ds(base + k * CHUNK, CHUNK)])
        pltpu.sync_copy(zerob, accd.at[pl.ds(base + k * CHUNK, CHUNK)])
    rem = OPT % CHUNK
    if rem:
        off = base + (OPT // CHUNK) * CHUNK
        pltpu.sync_copy(bufs.at[0].at[pl.ds(0, rem)], accf.at[pl.ds(off, rem)])
        pltpu.sync_copy(zerob.at[pl.ds(0, rem)], accd.at[pl.ds(off, rem)])
    plsc.subcore_barrier()

    NB = 5

    def _quint(g, carry):
        j0 = g * NB
        cidx = [colv.at[pl.ds((j0 + b) * CHUNK, CHUNK)] for b in range(NB)]
        ridx = [rowv.at[pl.ds((j0 + b) * CHUNK, CHUNK)] for b in range(NB)]
        ds = [pltpu.async_copy(x_hbm.at[cidx[b]], bufs.at[b], gsems[b])
              for b in range(NB)]
        ss = []
        for b in range(NB):
            ds[b].wait()
            sf = pltpu.make_async_copy(bufs.at[b], accf.at[ridx[b]], fsems[b])
            sf.start(add=True)
            sd = pltpu.make_async_copy(onesb, accd.at[ridx[b]], dsems[b])
            sd.start(add=True)
            ss += [sf, sd]
        for sc_ in ss:
            sc_.wait()
        return carry

    lax.fori_loop(0, NCH // NB, _quint, 0)
    plsc.subcore_barrier()

    pltpu.sync_copy(accf.at[pl.ds(base, OPT)], outf_hbm.at[c, pl.ds(base, OPT)])
    pltpu.sync_copy(accd.at[pl.ds(base, OPT)], outd_hbm.at[c, pl.ds(base, OPT)])


def _tc_body(pf0_ref, pf1_ref, pd0_ref, pd1_ref,
             w0_ref, b0_ref, w1_ref, b1_ref, y_ref):
    feat = (pf0_ref[0].astype(jnp.float32)
            + pf1_ref[0].astype(jnp.float32))
    deg = (pd0_ref[0] + pd1_ref[0])[:, :1]
    has_edge = jnp.minimum(deg, 1.0)
    m = feat / jnp.maximum(deg, 1.0)
    h = jnp.dot(m, w0_ref[...], preferred_element_type=jnp.float32)
    h = jnp.maximum(h + has_edge * b0_ref[...], 0.0)
    y_ref[...] = (jnp.dot(h, w1_ref[...], preferred_element_type=jnp.float32)
                  + b1_ref[...])


RB = 2000


def kernel(x, edge_index, W0, b0, W1, b1):
    colp = edge_index[1].astype(jnp.int32).reshape(NW, EPT)
    rowp = edge_index[0].astype(jnp.int32).reshape(NW, EPT)

    pf, pd = _sc_aggregate(x.astype(jnp.bfloat16), colp, rowp)

    y = pl.pallas_call(
        _tc_body,
        grid=(N // RB,),
        in_specs=[
            pl.BlockSpec((1, RB, D), lambda i: (0, i, 0)),
            pl.BlockSpec((1, RB, D), lambda i: (1, i, 0)),
            pl.BlockSpec((1, RB, DDEG), lambda i: (0, i, 0)),
            pl.BlockSpec((1, RB, DDEG), lambda i: (1, i, 0)),
            pl.BlockSpec((D, D), lambda i: (0, 0)),
            pl.BlockSpec((1, D), lambda i: (0, 0)),
            pl.BlockSpec((D, D), lambda i: (0, 0)),
            pl.BlockSpec((1, D), lambda i: (0, 0)),
        ],
        out_specs=pl.BlockSpec((RB, D), lambda i: (i, 0)),
        out_shape=jax.ShapeDtypeStruct((N, D), jnp.float32),
    )(pf, pf, pd, pd, W0, b0.reshape(1, D), W1, b1.reshape(1, D))
    return y
